# baseline (speedup 1.0000x reference)
import jax
import jax.numpy as jnp
from jax.experimental import pallas as pl
from jax.experimental.pallas import tpu as pltpu

_BN_EPS = 1e-5


def _fire_body(x_ref, sel_ref, gg_ref, gb_ref, fg_ref, fb_ref, w_ref,
               gbn_ref, y_ref, s0_ref, s1_ref):
    i = pl.program_id(0)
    x = x_ref[...]
    bt, HW, C = x.shape
    S = HW // 2

    s0_ref[pl.ds(i * bt, bt), :] = jnp.sum(x[:, :S, :], axis=1)
    s1_ref[pl.ds(i * bt, bt), :] = jnp.sum(x[:, S:, :], axis=1)

    @pl.when(i == pl.num_programs(0) - 1)
    def _():
        s0 = s0_ref[...]
        s1 = s1_ref[...]

        g = (s0 + s1) * (1.0 / HW)
        mu = jnp.mean(g, axis=0, keepdims=True)
        var = jnp.mean((g - mu) ** 2, axis=0, keepdims=True)
        gbn_ref[...] = ((g - mu) * jax.lax.rsqrt(var + _BN_EPS)
                        * gg_ref[...] + gb_ref[...])

        pooled = 0.5 * (1.0 / S) * (
            jnp.dot(sel_ref[0], s0, preferred_element_type=jnp.float32)
            + jnp.dot(sel_ref[1], s1, preferred_element_type=jnp.float32))
        bmu = jnp.mean(pooled, axis=0, keepdims=True)
        bvar = jnp.mean((pooled - bmu) ** 2, axis=0, keepdims=True)
        bn = ((pooled - bmu) * jax.lax.rsqrt(bvar + _BN_EPS)
              * fg_ref[...] + fb_ref[...])

        y_ref[...] = jnp.dot(bn, w_ref[...],
                             preferred_element_type=jnp.float32)


def _sample_negatives(sample_key, fgid, P):
    neg_mask = fgid[:, None] != fgid[None, :]
    logits = jnp.where(neg_mask, 0.0, -jnp.inf)
    keys = jax.random.split(sample_key, P)
    return jax.vmap(lambda k: jax.random.categorical(k, logits, axis=-1))(keys)


def kernel(feat_nhwc, fgid, bn_gamma, bn_beta, far_bn_gamma, far_bn_beta,
           cls_w_t, sample_key):
    B, H, W, C = feat_nhwc.shape
    HW = H * W
    P = 2
    x3 = feat_nhwc.reshape(B, HW, C)

    idx = _sample_negatives(sample_key, fgid, P)
    sel = jax.nn.one_hot(idx, B, dtype=jnp.float32)

    num_classes = cls_w_t.shape[1]
    bt = 8 if B % 8 == 0 else B

    gbn, y_far = pl.pallas_call(
        _fire_body,
        out_shape=(jax.ShapeDtypeStruct((B, C), jnp.float32),
                   jax.ShapeDtypeStruct((B, num_classes), jnp.float32)),
        grid=(B // bt,),
        in_specs=[
            pl.BlockSpec((bt, HW, C), lambda i: (i, 0, 0)),
            pl.BlockSpec((P, B, B), lambda i: (0, 0, 0)),
            pl.BlockSpec((1, C), lambda i: (0, 0)),
            pl.BlockSpec((1, C), lambda i: (0, 0)),
            pl.BlockSpec((1, C), lambda i: (0, 0)),
            pl.BlockSpec((1, C), lambda i: (0, 0)),
            pl.BlockSpec((C, num_classes), lambda i: (0, 0)),
        ],
        out_specs=(
            pl.BlockSpec((B, C), lambda i: (0, 0)),
            pl.BlockSpec((B, num_classes), lambda i: (0, 0)),
        ),
        scratch_shapes=[pltpu.VMEM((B, C), jnp.float32),
                        pltpu.VMEM((B, C), jnp.float32)],
        compiler_params=pltpu.CompilerParams(
            dimension_semantics=("arbitrary",),
            vmem_limit_bytes=48 * 1024 * 1024),
    )(x3, sel, bn_gamma, bn_beta, far_bn_gamma, far_bn_beta, cls_w_t)

    return gbn, y_far
